# baseline (speedup 1.0000x reference)
import functools

import jax
import jax.numpy as jnp
from jax import lax
from jax.experimental import pallas as pl
from jax.experimental.pallas import tpu as pltpu
from jax.experimental.pallas import tpu_sc as plsc

NUM_EXPERTS_K = 64
TOPK_K = 8
ROW_BLOCK = 1024
N_TOKENS = 16384
NC, NS, LANES = 2, 16, 16
NW = NC * NS
RPW = N_TOKENS // NW


def _matmul_body(x_ref, w_ref, s_ref):
    s_ref[...] = jnp.dot(
        x_ref[...], w_ref[...], preferred_element_type=jnp.float32
    )


def _tc_scores(x_flat, W):
    N, D = x_flat.shape
    E = W.shape[1]
    return pl.pallas_call(
        _matmul_body,
        grid=(N // ROW_BLOCK,),
        in_specs=[
            pl.BlockSpec((ROW_BLOCK, D), lambda i: (i, 0)),
            pl.BlockSpec((D, E), lambda i: (0, 0)),
        ],
        out_specs=pl.BlockSpec((ROW_BLOCK, E), lambda i: (i, 0)),
        out_shape=jax.ShapeDtypeStruct((N, E), jnp.float32),
    )(x_flat, W)


@functools.partial(
    pl.kernel,
    out_type=[
        jax.ShapeDtypeStruct((N_TOKENS, LANES), jnp.float32),
        jax.ShapeDtypeStruct((N_TOKENS, LANES), jnp.int32),
        jax.ShapeDtypeStruct((N_TOKENS, NUM_EXPERTS_K), jnp.float32),
    ],
    mesh=plsc.VectorSubcoreMesh(core_axis_name="c", subcore_axis_name="s"),
    compiler_params=pltpu.CompilerParams(
        needs_layout_passes=False, use_tc_tiling_on_sc=False
    ),
    scratch_types=[
        pltpu.VMEM((RPW, NUM_EXPERTS_K), jnp.float32),
        pltpu.VMEM((RPW, LANES), jnp.float32),
        pltpu.VMEM((RPW, LANES), jnp.int32),
        pltpu.VMEM((RPW, NUM_EXPERTS_K), jnp.float32),
    ],
)
def _sc_router(scores_hbm, wts_hbm, idx_hbm, mask_hbm, s_v, w_v, i_v, m_v):
    wid = lax.axis_index("s") * NC + lax.axis_index("c")
    base = wid * RPW
    pltpu.sync_copy(scores_hbm.at[pl.ds(base, RPW)], s_v)

    lane = lax.iota(jnp.int32, 16)
    lane_lt8 = lane < TOPK_K
    flane = lane.astype(jnp.float32)
    chunk_idx = [lane + LANES * c for c in range(NUM_EXPERTS_K // LANES)]

    def row_body(r, _):
        s_orig = [
            s_v[r, pl.ds(c * LANES, LANES)]
            for c in range(NUM_EXPERTS_K // LANES)
        ]
        cur = list(s_orig)
        vals = []
        idxs = []
        for _k in range(TOPK_K):
            m01 = jnp.maximum(cur[0], cur[1])
            m23 = jnp.maximum(cur[2], cur[3])
            m = jnp.max(jnp.maximum(m01, m23))
            cands = [
                jnp.where(cur[c] == m, chunk_idx[c], NUM_EXPERTS_K)
                for c in range(4)
            ]
            c01 = jnp.minimum(cands[0], cands[1])
            c23 = jnp.minimum(cands[2], cands[3])
            pick = jnp.min(jnp.minimum(c01, c23))
            cur = [
                jnp.where(chunk_idx[c] == pick, -jnp.inf, cur[c])
                for c in range(4)
            ]
            vals.append(m)
            idxs.append(pick)

        vvec = jnp.zeros((16,), jnp.float32)
        ivec = jnp.zeros((16,), jnp.int32)
        for j in range(TOPK_K):
            vvec = jnp.where(lane == j, vals[j], vvec)
            ivec = jnp.where(lane == j, idxs[j], ivec)

        m0 = vals[0]
        e_all = [jnp.exp(s_orig[c] - m0) for c in range(4)]
        total = jnp.sum(e_all[0] + e_all[1] + e_all[2] + e_all[3])
        w16 = jnp.exp(vvec - m0)
        s8 = jnp.sum(jnp.where(lane_lt8, w16, 0.0))
        w_v[r, :] = w16 / (s8 + 1e-8 * total)
        i_v[r, :] = ivec

        v8 = vals[TOPK_K - 1]
        i8 = idxs[TOPK_K - 1]
        for c in range(4):
            hit = (s_orig[c] > v8) | ((s_orig[c] == v8) & (chunk_idx[c] <= i8))
            m_v[r, pl.ds(c * LANES, LANES)] = jnp.where(hit, 1.0, 0.0)
        return 0

    lax.fori_loop(0, RPW, row_body, 0)

    pltpu.sync_copy(w_v, wts_hbm.at[pl.ds(base, RPW)])
    pltpu.sync_copy(i_v, idx_hbm.at[pl.ds(base, RPW)])
    pltpu.sync_copy(m_v, mask_hbm.at[pl.ds(base, RPW)])


@functools.partial(jax.jit, static_argnames=())
def kernel(x, W):
    B, S, D = x.shape
    N = B * S
    E = W.shape[1]
    x_flat = x.reshape(N, D)
    scores = _tc_scores(x_flat, W)
    wts16, idx16, mask = _sc_router(scores)
    return wts16[:, :TOPK_K], idx16[:, :TOPK_K], mask.reshape(B, S, E)


# speedup vs baseline: 2.2392x; 2.2392x over previous
import functools

import jax
import jax.numpy as jnp
from jax.experimental import pallas as pl
from jax.experimental.pallas import tpu as pltpu

NUM_EXPERTS_K = 64
TOPK_K = 8
ROW_BLOCK = 1024


def _router_body(x_ref, w_ref, wts_ref, idx_ref, mask_ref):
    scores = jnp.dot(x_ref[...], w_ref[...], preferred_element_type=jnp.float32)

    fiota = jax.lax.broadcasted_iota(
        jnp.int32, scores.shape, 1
    ).astype(jnp.float32)
    cur = scores
    mask = jnp.zeros_like(scores)
    vals = []
    idxs = []
    for _ in range(TOPK_K):
        mj = jnp.max(cur, axis=-1, keepdims=True)
        is_max = cur == mj
        ij = jnp.min(
            jnp.where(is_max, fiota, float(NUM_EXPERTS_K)), axis=-1, keepdims=True
        )
        onehot = fiota == ij
        mask = jnp.where(onehot, 1.0, mask)
        cur = jnp.where(onehot, -jnp.inf, cur)
        vals.append(mj)
        idxs.append(ij)
    top_vals = jnp.concatenate(vals, axis=1)
    top_fidx = jnp.concatenate(idxs, axis=1)
    e8 = jnp.exp(top_vals - vals[0])
    wts_ref[...] = e8 / (jnp.sum(e8, axis=1, keepdims=True) * (1.0 + 1e-8))
    idx_ref[...] = top_fidx.astype(jnp.int32)
    mask_ref[...] = mask


@functools.partial(jax.jit, static_argnames=())
def kernel(x, W):
    B, S, D = x.shape
    N = B * S
    E = W.shape[1]
    x_flat = x.reshape(N, D)
    grid = (N // ROW_BLOCK,)
    wts, idx, mask = pl.pallas_call(
        _router_body,
        grid=grid,
        in_specs=[
            pl.BlockSpec((ROW_BLOCK, D), lambda i: (i, 0)),
            pl.BlockSpec((D, E), lambda i: (0, 0)),
        ],
        out_specs=[
            pl.BlockSpec((ROW_BLOCK, TOPK_K), lambda i: (i, 0)),
            pl.BlockSpec((ROW_BLOCK, TOPK_K), lambda i: (i, 0)),
            pl.BlockSpec((ROW_BLOCK, E), lambda i: (i, 0)),
        ],
        out_shape=[
            jax.ShapeDtypeStruct((N, TOPK_K), jnp.float32),
            jax.ShapeDtypeStruct((N, TOPK_K), jnp.int32),
            jax.ShapeDtypeStruct((N, E), jnp.float32),
        ],
    )(x_flat, W)
    return wts, idx, mask.reshape(B, S, E)


# speedup vs baseline: 2.2394x; 1.0001x over previous
import functools

import jax
import jax.numpy as jnp
from jax.experimental import pallas as pl

NUM_EXPERTS_K = 64
TOPK_K = 8
ROW_BLOCK = 1024


def _router_body(x_ref, w_ref, wts_ref, idx_ref, mask_ref):
    scores = jnp.dot(x_ref[...], w_ref[...], preferred_element_type=jnp.float32)

    fiota = jax.lax.broadcasted_iota(
        jnp.int32, scores.shape, 1
    ).astype(jnp.float32)
    cur = scores
    vals = []
    idxs = []
    for _ in range(TOPK_K):
        mj = jnp.max(cur, axis=-1, keepdims=True)
        is_max = cur == mj
        ij = jnp.min(
            jnp.where(is_max, fiota, float(NUM_EXPERTS_K)), axis=-1, keepdims=True
        )
        cur = jnp.where(fiota == ij, -jnp.inf, cur)
        vals.append(mj)
        idxs.append(ij)
    mask = jnp.where(cur == -jnp.inf, 1.0, 0.0)
    top_vals = jnp.concatenate(vals, axis=1)
    top_fidx = jnp.concatenate(idxs, axis=1)
    e8 = jnp.exp(top_vals - vals[0])
    wts_ref[...] = e8 / (jnp.sum(e8, axis=1, keepdims=True) * (1.0 + 1e-8))
    idx_ref[...] = top_fidx.astype(jnp.int32)
    mask_ref[...] = mask


@functools.partial(jax.jit, static_argnames=())
def kernel(x, W):
    B, S, D = x.shape
    N = B * S
    E = W.shape[1]
    x_flat = x.reshape(N, D)
    grid = (N // ROW_BLOCK,)
    wts, idx, mask = pl.pallas_call(
        _router_body,
        grid=grid,
        in_specs=[
            pl.BlockSpec((ROW_BLOCK, D), lambda i: (i, 0)),
            pl.BlockSpec((D, E), lambda i: (0, 0)),
        ],
        out_specs=[
            pl.BlockSpec((ROW_BLOCK, TOPK_K), lambda i: (i, 0)),
            pl.BlockSpec((ROW_BLOCK, TOPK_K), lambda i: (i, 0)),
            pl.BlockSpec((ROW_BLOCK, E), lambda i: (i, 0)),
        ],
        out_shape=[
            jax.ShapeDtypeStruct((N, TOPK_K), jnp.float32),
            jax.ShapeDtypeStruct((N, TOPK_K), jnp.int32),
            jax.ShapeDtypeStruct((N, E), jnp.float32),
        ],
    )(x_flat, W)
    return wts, idx, mask.reshape(B, S, E)
